# baseline (speedup 1.0000x reference)
import functools

import jax
import jax.numpy as jnp
from jax import lax
from jax.experimental import pallas as pl
from jax.experimental.pallas import tpu as pltpu
from jax.experimental.pallas import tpu_sc as plsc

NC = 2
NS = 16
NW = NC * NS

B = 4096
S = 50
SP = 56
D = 64
BPW = B // NW
GB = 2
NG = BPW // GB
NBUF = 4
LOOKAHEAD = 2

TB = 8192


def _transpose_pad(table_t):
    v = table_t.shape[1]
    grid = (v + TB - 1) // TB

    def body(in_ref, out_ref):
        out_ref[:, 0:D] = jnp.transpose(in_ref[...], (1, 0))

    return pl.pallas_call(
        body,
        grid=(grid,),
        in_specs=[pl.BlockSpec((D, TB), lambda j: (0, j))],
        out_specs=pl.BlockSpec((TB, 2 * D), lambda j: (j, 0)),
        out_shape=jax.ShapeDtypeStruct((v, 2 * D), jnp.float32),
    )(table_t)


def _make_gather(num_embeddings):
    mesh = plsc.VectorSubcoreMesh(
        core_axis_name="c", subcore_axis_name="s",
        num_cores=NC, num_subcores=NS)

    @functools.partial(
        pl.kernel,
        out_type=jax.ShapeDtypeStruct((B * SP, 2 * D), jnp.float32),
        mesh=mesh,
        scratch_types=(
            [pltpu.VMEM((BPW, S), jnp.int32)]
            + [pltpu.VMEM((GB * SP, 2 * D), jnp.float32)
               for _ in range(NBUF)]
            + [pltpu.SemaphoreType.DMA for _ in range(NBUF)]
            + [pltpu.SemaphoreType.DMA for _ in range(NBUF)]
        ),
        compiler_params=pltpu.CompilerParams(needs_layout_passes=False),
    )
    def gather(idx_hbm, tpad_hbm, out_hbm, idx_v, *scr):
        rows = scr[0:NBUF]
        gsem = scr[NBUF:2 * NBUF]
        osem = scr[2 * NBUF:3 * NBUF]
        wid = lax.axis_index("s") * NC + lax.axis_index("c")

        pltpu.sync_copy(idx_hbm.at[wid], idx_v)

        def fire(g, b):
            for i in range(GB):
                pltpu.async_copy(
                    tpad_hbm.at[idx_v.at[g * GB + i]],
                    rows[b].at[pl.ds(i * SP, S)], gsem[b])

        def wait_gather(b):
            for i in range(GB):
                pltpu.make_async_copy(
                    tpad_hbm.at[idx_v.at[0]],
                    rows[b].at[pl.ds(i * SP, S)], gsem[b]).wait()

        def store(g, b):
            pltpu.async_copy(
                rows[b],
                out_hbm.at[pl.ds((wid * BPW + g * GB) * SP, GB * SP)],
                osem[b])

        def wait_store(b):
            pltpu.make_async_copy(
                rows[b],
                out_hbm.at[pl.ds(0, GB * SP)], osem[b]).wait()

        for g0 in range(LOOKAHEAD):
            fire(g0, g0 % NBUF)

        @pl.loop(0, NG, step=NBUF)
        def _(outer):
            for b in range(NBUF):
                g = outer + b
                wait_gather(b)
                store(g, b)
                nxt = (b + LOOKAHEAD) % NBUF

                @pl.when(outer + b + LOOKAHEAD < NG)
                def _():
                    @pl.when(outer + b + LOOKAHEAD >= NBUF)
                    def _():
                        wait_store(nxt)
                    fire(g + LOOKAHEAD, nxt)

        for b in range(NBUF):
            wait_store(b)

    return gather


def kernel(token_ids, embedding_matrix):
    n, s = token_ids.shape
    idx = token_ids.astype(jnp.int32).reshape(NW, BPW, S)
    tpad = _transpose_pad(embedding_matrix.T)
    out = _make_gather(embedding_matrix.shape[0])(idx, tpad)
    return out.reshape(n, SP, 2 * D)[:, :s, :D]


# speedup vs baseline: 1.0488x; 1.0488x over previous
import functools

import jax
import jax.numpy as jnp
from jax import lax
from jax.experimental import pallas as pl
from jax.experimental.pallas import tpu as pltpu
from jax.experimental.pallas import tpu_sc as plsc

NC = 2
NS = 16
NW = NC * NS

B = 4096
S = 50
SP = 56
D = 64
BPW = B // NW
GB = 2
NG = BPW // GB
NBUF = 4
LOOKAHEAD = 2

TB = 16384


def _transpose_pad(table_t):
    v = table_t.shape[1]
    grid = (v + TB - 1) // TB

    def body(in_ref, out_ref):
        out_ref[:, 0:D] = jnp.transpose(in_ref[...], (1, 0))

    return pl.pallas_call(
        body,
        grid=(grid,),
        in_specs=[pl.BlockSpec((D, TB), lambda j: (0, j))],
        out_specs=pl.BlockSpec((TB, 2 * D), lambda j: (j, 0)),
        out_shape=jax.ShapeDtypeStruct((v, 2 * D), jnp.float32),
    )(table_t)


def _make_gather(num_embeddings):
    mesh = plsc.VectorSubcoreMesh(
        core_axis_name="c", subcore_axis_name="s",
        num_cores=NC, num_subcores=NS)

    @functools.partial(
        pl.kernel,
        out_type=jax.ShapeDtypeStruct((B * SP, 2 * D), jnp.float32),
        mesh=mesh,
        scratch_types=(
            [pltpu.VMEM((BPW, S), jnp.int32)]
            + [pltpu.VMEM((GB * SP, 2 * D), jnp.float32)
               for _ in range(NBUF)]
            + [pltpu.SemaphoreType.DMA for _ in range(NBUF)]
            + [pltpu.SemaphoreType.DMA for _ in range(NBUF)]
        ),
        compiler_params=pltpu.CompilerParams(needs_layout_passes=False),
    )
    def gather(idx_hbm, tpad_hbm, out_hbm, idx_v, *scr):
        rows = scr[0:NBUF]
        gsem = scr[NBUF:2 * NBUF]
        osem = scr[2 * NBUF:3 * NBUF]
        wid = lax.axis_index("s") * NC + lax.axis_index("c")

        pltpu.sync_copy(idx_hbm.at[wid], idx_v)

        def fire(g, b):
            for i in range(GB):
                pltpu.async_copy(
                    tpad_hbm.at[idx_v.at[g * GB + i]],
                    rows[b].at[pl.ds(i * SP, S)], gsem[b])

        def wait_gather(b):
            for i in range(GB):
                pltpu.make_async_copy(
                    tpad_hbm.at[idx_v.at[0]],
                    rows[b].at[pl.ds(i * SP, S)], gsem[b]).wait()

        def store(g, b):
            pltpu.async_copy(
                rows[b],
                out_hbm.at[pl.ds((wid * BPW + g * GB) * SP, GB * SP)],
                osem[b])

        def wait_store(b):
            pltpu.make_async_copy(
                rows[b],
                out_hbm.at[pl.ds(0, GB * SP)], osem[b]).wait()

        for g0 in range(LOOKAHEAD):
            fire(g0, g0 % NBUF)

        @pl.loop(0, NG, step=NBUF)
        def _(outer):
            for b in range(NBUF):
                g = outer + b
                wait_gather(b)
                store(g, b)
                nxt = (b + LOOKAHEAD) % NBUF

                @pl.when(outer + b + LOOKAHEAD < NG)
                def _():
                    @pl.when(outer + b + LOOKAHEAD >= NBUF)
                    def _():
                        wait_store(nxt)
                    fire(g + LOOKAHEAD, nxt)

        for b in range(NBUF):
            wait_store(b)

    return gather


def kernel(token_ids, embedding_matrix):
    n, s = token_ids.shape
    idx = token_ids.astype(jnp.int32).reshape(NW, BPW, S)
    tpad = _transpose_pad(embedding_matrix.T)
    out = _make_gather(embedding_matrix.shape[0])(idx, tpad)
    return out.reshape(n, SP, 2 * D)[:, :s, :D]


# speedup vs baseline: 1.0673x; 1.0176x over previous
import functools

import jax
import jax.numpy as jnp
from jax import lax
from jax.experimental import pallas as pl
from jax.experimental.pallas import tpu as pltpu
from jax.experimental.pallas import tpu_sc as plsc

NC = 2
NS = 16
NW = NC * NS

B = 4096
S = 50
SP = 56
D = 64
BPW = B // NW
GB = 2
NG = BPW // GB
NBUF = 4
LOOKAHEAD = 2

TB = 32768


def _transpose_pad(table_t):
    v = table_t.shape[1]
    grid = (v + TB - 1) // TB

    def body(in_ref, out_ref):
        out_ref[:, 0:D] = jnp.transpose(in_ref[...], (1, 0))

    return pl.pallas_call(
        body,
        grid=(grid,),
        in_specs=[pl.BlockSpec((D, TB), lambda j: (0, j))],
        out_specs=pl.BlockSpec((TB, 2 * D), lambda j: (j, 0)),
        out_shape=jax.ShapeDtypeStruct((v, 2 * D), jnp.float32),
    )(table_t)


def _make_gather(num_embeddings):
    mesh = plsc.VectorSubcoreMesh(
        core_axis_name="c", subcore_axis_name="s",
        num_cores=NC, num_subcores=NS)

    @functools.partial(
        pl.kernel,
        out_type=jax.ShapeDtypeStruct((B * SP, 2 * D), jnp.float32),
        mesh=mesh,
        scratch_types=(
            [pltpu.VMEM((BPW, S), jnp.int32)]
            + [pltpu.VMEM((GB * SP, 2 * D), jnp.float32)
               for _ in range(NBUF)]
            + [pltpu.SemaphoreType.DMA for _ in range(NBUF)]
            + [pltpu.SemaphoreType.DMA for _ in range(NBUF)]
        ),
        compiler_params=pltpu.CompilerParams(needs_layout_passes=False),
    )
    def gather(idx_hbm, tpad_hbm, out_hbm, idx_v, *scr):
        rows = scr[0:NBUF]
        gsem = scr[NBUF:2 * NBUF]
        osem = scr[2 * NBUF:3 * NBUF]
        wid = lax.axis_index("s") * NC + lax.axis_index("c")

        pltpu.sync_copy(idx_hbm.at[wid], idx_v)

        def fire(g, b):
            for i in range(GB):
                pltpu.async_copy(
                    tpad_hbm.at[idx_v.at[g * GB + i]],
                    rows[b].at[pl.ds(i * SP, S)], gsem[b])

        def wait_gather(b):
            for i in range(GB):
                pltpu.make_async_copy(
                    tpad_hbm.at[idx_v.at[0]],
                    rows[b].at[pl.ds(i * SP, S)], gsem[b]).wait()

        def store(g, b):
            pltpu.async_copy(
                rows[b],
                out_hbm.at[pl.ds((wid * BPW + g * GB) * SP, GB * SP)],
                osem[b])

        def wait_store(b):
            pltpu.make_async_copy(
                rows[b],
                out_hbm.at[pl.ds(0, GB * SP)], osem[b]).wait()

        for g0 in range(LOOKAHEAD):
            fire(g0, g0 % NBUF)

        @pl.loop(0, NG, step=NBUF)
        def _(outer):
            for b in range(NBUF):
                g = outer + b
                wait_gather(b)
                store(g, b)
                nxt = (b + LOOKAHEAD) % NBUF

                @pl.when(outer + b + LOOKAHEAD < NG)
                def _():
                    @pl.when(outer + b + LOOKAHEAD >= NBUF)
                    def _():
                        wait_store(nxt)
                    fire(g + LOOKAHEAD, nxt)

        for b in range(NBUF):
            wait_store(b)

    return gather


def kernel(token_ids, embedding_matrix):
    n, s = token_ids.shape
    idx = token_ids.astype(jnp.int32).reshape(NW, BPW, S)
    tpad = _transpose_pad(embedding_matrix.T)
    out = _make_gather(embedding_matrix.shape[0])(idx, tpad)
    return out.reshape(n, SP, 2 * D)[:, :s, :D]
